# baseline (speedup 1.0000x reference)
import functools

import jax
import jax.numpy as jnp
from jax import lax
from jax.experimental import pallas as pl
from jax.experimental.pallas import tpu as pltpu
from jax.experimental.pallas import tpu_sc as plsc

_EPS = 1e-12
_NC = 2
_NS = 16
_NW = _NC * _NS
_GROUP = 128


def _sc_gather(ids3, table):
    nw, ngroups, group = ids3.shape
    v, dp = table.shape
    n = nw * ngroups * group
    per_w = ngroups * group
    mesh = plsc.VectorSubcoreMesh(core_axis_name="c", subcore_axis_name="s")

    @functools.partial(
        pl.kernel,
        mesh=mesh,
        out_type=jax.ShapeDtypeStruct((n, dp), jnp.float32),
        scratch_types=[
            pltpu.VMEM((ngroups, group), jnp.int32),
            pltpu.VMEM((2, group, dp), jnp.float32),
            pltpu.SemaphoreType.DMA,
            pltpu.SemaphoreType.DMA,
        ],
        compiler_params=pltpu.CompilerParams(use_tc_tiling_on_sc=False),
    )
    def k(idx_hbm, table_hbm, out_hbm, idx_v, rows_v, sem0, sem1):
        wid = lax.axis_index("s") * _NC + lax.axis_index("c")
        base = wid * per_w
        pltpu.sync_copy(idx_hbm.at[wid], idx_v)

        pltpu.async_copy(table_hbm.at[idx_v.at[0]], rows_v.at[0], sem0).wait()

        def body(jj, carry):
            j0 = jj * 2
            pltpu.async_copy(table_hbm.at[idx_v.at[j0 + 1]], rows_v.at[1], sem1)
            off0 = pl.multiple_of(base + j0 * group, group)
            pltpu.sync_copy(rows_v.at[0], out_hbm.at[pl.ds(off0, group)])
            cp1 = pltpu.make_async_copy(table_hbm.at[idx_v.at[j0 + 1]],
                                        rows_v.at[1], sem1)
            cp1.wait()
            is_last = jj == (ngroups // 2 - 1)

            @pl.when(jnp.logical_not(is_last))
            def _():
                pltpu.async_copy(table_hbm.at[idx_v.at[j0 + 2]], rows_v.at[0],
                                 sem0)

            off1 = pl.multiple_of(base + (j0 + 1) * group, group)
            pltpu.sync_copy(rows_v.at[1], out_hbm.at[pl.ds(off1, group)])

            @pl.when(jnp.logical_not(is_last))
            def _():
                pltpu.make_async_copy(table_hbm.at[idx_v.at[j0 + 2]],
                                      rows_v.at[0], sem0).wait()

            return carry

        lax.fori_loop(0, ngroups // 2, body, 0)

    return k(ids3, table)


def _fmt_body(wet_ref, o_ref):
    x = wet_ref[...]
    o_ref[:, : x.shape[0]] = x.T


def _tc_format(wet, bi=20480):
    d, v = wet.shape
    nb = (v + bi - 1) // bi
    vp = nb * bi
    return pl.pallas_call(
        _fmt_body,
        grid=(nb,),
        in_specs=[pl.BlockSpec((d, bi), lambda k: (0, k))],
        out_specs=pl.BlockSpec((bi, 128), lambda k: (k, 0)),
        out_shape=jax.ShapeDtypeStruct((vp, 128), jnp.float32),
    )(wet)


def _lane_sum(x):
    while x.shape[-1] > 1:
        h = x.shape[-1] // 2
        x = x[..., :h] + x[..., h:]
    return x


def _ln_body(we_ref, tt_ref, pe_ref, tte_ref, g_ref, b_ref, o_ref):
    bb, l, d = o_ref.shape
    we = we_ref[...].reshape(bb, l, we_ref.shape[-1])[:, :, :d]
    ttf = tt_ref[...].astype(jnp.float32)
    pe = pe_ref[...]
    tte = tte_ref[...]
    te = tte[0][None, None, :] + ttf[..., None] * (tte[1] - tte[0])[None, None, :]
    emb = we + pe[None] + te
    mean = jnp.mean(emb, axis=-1, keepdims=True)
    c = emb - mean
    var = jnp.mean(c * c, axis=-1, keepdims=True)
    inv = lax.rsqrt(var + _EPS)
    o_ref[...] = c * inv * g_ref[...][None, None, :] + b_ref[...][None, None, :]


def _tc_layernorm(we_pad, tt, pe, tte, gamma, beta):
    b, l = tt.shape
    d = pe.shape[-1]
    bb = 32
    grid = (b // bb,)
    return pl.pallas_call(
        _ln_body,
        grid=grid,
        in_specs=[
            pl.BlockSpec((bb * l, 128), lambda i: (i, 0)),
            pl.BlockSpec((bb, l), lambda i: (i, 0)),
            pl.BlockSpec((l, d), lambda i: (0, 0)),
            pl.BlockSpec((2, d), lambda i: (0, 0)),
            pl.BlockSpec((d,), lambda i: (0,)),
            pl.BlockSpec((d,), lambda i: (0,)),
        ],
        out_specs=pl.BlockSpec((bb, l, d), lambda i: (i, 0, 0)),
        out_shape=jax.ShapeDtypeStruct((b, l, d), jnp.float32),
    )(we_pad, tt, pe, tte, gamma, beta)


def kernel(input_ids, token_type_ids, word_emb, pos_emb, tok_type_emb, gamma, beta):
    b, l = input_ids.shape
    v, d = word_emb.shape
    n = b * l
    per_w = n // _NW
    ngroups = per_w // _GROUP
    ids = input_ids.astype(jnp.int32)
    tt = token_type_ids.astype(jnp.int32)
    pe = pos_emb[:l]
    table_pad = _tc_format(word_emb.T)
    nsplit = 2
    bh = b // nsplit
    outs = []
    for h in range(nsplit):
        ids3 = ids[h * bh:(h + 1) * bh].reshape(_NW, ngroups // nsplit, _GROUP)
        we_pad = _sc_gather(ids3, table_pad)
        outs.append(_tc_layernorm(we_pad, tt[h * bh:(h + 1) * bh], pe,
                                  tok_type_emb, gamma, beta))
    return jnp.concatenate(outs, axis=0)


# speedup vs baseline: 1.0705x; 1.0705x over previous
import functools

import jax
import jax.numpy as jnp
from jax import lax
from jax.experimental import pallas as pl
from jax.experimental.pallas import tpu as pltpu
from jax.experimental.pallas import tpu_sc as plsc

_EPS = 1e-12
_NC = 2
_NS = 16
_NW = _NC * _NS
_GROUP = 128


def _sc_gather(ids3, table):
    nw, ngroups, group = ids3.shape
    v, dp = table.shape
    n = nw * ngroups * group
    per_w = ngroups * group
    mesh = plsc.VectorSubcoreMesh(core_axis_name="c", subcore_axis_name="s")

    nbuf = 4
    nmain = ngroups // nbuf
    ntail = ngroups % nbuf

    @functools.partial(
        pl.kernel,
        mesh=mesh,
        out_type=jax.ShapeDtypeStruct((n, dp), jnp.float32),
        scratch_types=[
            pltpu.VMEM((ngroups, group), jnp.int32),
            pltpu.VMEM((nbuf, group, dp), jnp.float32),
            [pltpu.SemaphoreType.DMA] * nbuf,
        ],
        compiler_params=pltpu.CompilerParams(use_tc_tiling_on_sc=False),
    )
    def k(idx_hbm, table_hbm, out_hbm, idx_v, rows_v, sems):
        wid = lax.axis_index("s") * _NC + lax.axis_index("c")
        base = wid * per_w
        pltpu.sync_copy(idx_hbm.at[wid], idx_v)

        for s in range(nbuf):
            pltpu.async_copy(table_hbm.at[idx_v.at[s]], rows_v.at[s], sems[s])

        def body(jj, carry):
            j0 = jj * nbuf
            for s in range(nbuf):
                g = j0 + s
                pltpu.make_async_copy(table_hbm.at[idx_v.at[g]], rows_v.at[s],
                                      sems[s]).wait()
                off = pl.multiple_of(base + g * group, group)
                pltpu.sync_copy(rows_v.at[s], out_hbm.at[pl.ds(off, group)])

                @pl.when(g + nbuf < ngroups)
                def _():
                    pltpu.async_copy(table_hbm.at[idx_v.at[g + nbuf]],
                                     rows_v.at[s], sems[s])

            return carry

        lax.fori_loop(0, nmain, body, 0)

        for t in range(ntail):
            g = nmain * nbuf + t
            pltpu.make_async_copy(table_hbm.at[idx_v.at[g]], rows_v.at[t],
                                  sems[t]).wait()
            off = pl.multiple_of(base + g * group, group)
            pltpu.sync_copy(rows_v.at[t], out_hbm.at[pl.ds(off, group)])

    return k(ids3, table)


def _fmt_body(wet_ref, o_ref):
    x = wet_ref[...]
    o_ref[:, : x.shape[0]] = x.T


def _tc_format(wet, bi=20480):
    d, v = wet.shape
    nb = (v + bi - 1) // bi
    vp = nb * bi
    return pl.pallas_call(
        _fmt_body,
        grid=(nb,),
        in_specs=[pl.BlockSpec((d, bi), lambda k: (0, k))],
        out_specs=pl.BlockSpec((bi, 128), lambda k: (k, 0)),
        out_shape=jax.ShapeDtypeStruct((vp, 128), jnp.float32),
    )(wet)


def _lane_sum(x):
    while x.shape[-1] > 1:
        h = x.shape[-1] // 2
        x = x[..., :h] + x[..., h:]
    return x


def _ln_body(we_ref, tt_ref, pe_ref, tte_ref, g_ref, b_ref, o_ref):
    bb, l, d = o_ref.shape
    we = we_ref[...].reshape(bb, l, we_ref.shape[-1])[:, :, :d]
    ttf = tt_ref[...].astype(jnp.float32)
    pe = pe_ref[...]
    tte = tte_ref[...]
    te = tte[0][None, None, :] + ttf[..., None] * (tte[1] - tte[0])[None, None, :]
    emb = we + pe[None] + te
    mean = jnp.mean(emb, axis=-1, keepdims=True)
    c = emb - mean
    var = jnp.mean(c * c, axis=-1, keepdims=True)
    inv = lax.rsqrt(var + _EPS)
    o_ref[...] = c * inv * g_ref[...][None, None, :] + b_ref[...][None, None, :]


def _tc_layernorm(we_pad, tt, pe, tte, gamma, beta):
    b, l = tt.shape
    d = pe.shape[-1]
    bb = 32
    grid = (b // bb,)
    return pl.pallas_call(
        _ln_body,
        grid=grid,
        in_specs=[
            pl.BlockSpec((bb * l, 128), lambda i: (i, 0)),
            pl.BlockSpec((bb, l), lambda i: (i, 0)),
            pl.BlockSpec((l, d), lambda i: (0, 0)),
            pl.BlockSpec((2, d), lambda i: (0, 0)),
            pl.BlockSpec((d,), lambda i: (0,)),
            pl.BlockSpec((d,), lambda i: (0,)),
        ],
        out_specs=pl.BlockSpec((bb, l, d), lambda i: (i, 0, 0)),
        out_shape=jax.ShapeDtypeStruct((b, l, d), jnp.float32),
    )(we_pad, tt, pe, tte, gamma, beta)


def kernel(input_ids, token_type_ids, word_emb, pos_emb, tok_type_emb, gamma, beta):
    b, l = input_ids.shape
    v, d = word_emb.shape
    n = b * l
    per_w = n // _NW
    ngroups = per_w // _GROUP
    ids = input_ids.astype(jnp.int32)
    table_pad = _tc_format(word_emb.T)
    ids3 = ids.reshape(_NW, ngroups, _GROUP)
    we_pad = _sc_gather(ids3, table_pad)
    return _tc_layernorm(we_pad, token_type_ids.astype(jnp.int32),
                         pos_emb[:l], tok_type_emb, gamma, beta)


# speedup vs baseline: 1.0717x; 1.0011x over previous
import functools

import jax
import jax.numpy as jnp
from jax import lax
from jax.experimental import pallas as pl
from jax.experimental.pallas import tpu as pltpu
from jax.experimental.pallas import tpu_sc as plsc

_EPS = 1e-12
_NC = 2
_NS = 16
_NW = _NC * _NS
_GROUP = 128


def _sc_gather(ids3, table):
    nw, ngroups, group = ids3.shape
    v, dp = table.shape
    n = nw * ngroups * group
    per_w = ngroups * group
    mesh = plsc.VectorSubcoreMesh(core_axis_name="c", subcore_axis_name="s")

    nbuf = 6
    nmain = ngroups // nbuf
    ntail = ngroups % nbuf

    @functools.partial(
        pl.kernel,
        mesh=mesh,
        out_type=jax.ShapeDtypeStruct((n, dp), jnp.float32),
        scratch_types=[
            pltpu.VMEM((ngroups, group), jnp.int32),
            pltpu.VMEM((nbuf, group, dp), jnp.float32),
            [pltpu.SemaphoreType.DMA] * nbuf,
        ],
        compiler_params=pltpu.CompilerParams(use_tc_tiling_on_sc=False),
    )
    def k(idx_hbm, table_hbm, out_hbm, idx_v, rows_v, sems):
        wid = lax.axis_index("s") * _NC + lax.axis_index("c")
        base = wid * per_w
        pltpu.sync_copy(idx_hbm.at[wid], idx_v)

        for s in range(nbuf):
            pltpu.async_copy(table_hbm.at[idx_v.at[s]], rows_v.at[s], sems[s])

        def body(jj, carry):
            j0 = jj * nbuf
            for s in range(nbuf):
                g = j0 + s
                pltpu.make_async_copy(table_hbm.at[idx_v.at[g]], rows_v.at[s],
                                      sems[s]).wait()
                off = pl.multiple_of(base + g * group, group)
                pltpu.sync_copy(rows_v.at[s], out_hbm.at[pl.ds(off, group)])

                @pl.when(g + nbuf < ngroups)
                def _():
                    pltpu.async_copy(table_hbm.at[idx_v.at[g + nbuf]],
                                     rows_v.at[s], sems[s])

            return carry

        lax.fori_loop(0, nmain, body, 0)

        for t in range(ntail):
            g = nmain * nbuf + t
            pltpu.make_async_copy(table_hbm.at[idx_v.at[g]], rows_v.at[t],
                                  sems[t]).wait()
            off = pl.multiple_of(base + g * group, group)
            pltpu.sync_copy(rows_v.at[t], out_hbm.at[pl.ds(off, group)])

    return k(ids3, table)


def _fmt_body(wet_ref, o_ref):
    x = wet_ref[...]
    o_ref[:, : x.shape[0]] = x.T


def _tc_format(wet, bi=20480):
    d, v = wet.shape
    nb = (v + bi - 1) // bi
    vp = nb * bi
    return pl.pallas_call(
        _fmt_body,
        grid=(nb,),
        in_specs=[pl.BlockSpec((d, bi), lambda k: (0, k))],
        out_specs=pl.BlockSpec((bi, 128), lambda k: (k, 0)),
        out_shape=jax.ShapeDtypeStruct((vp, 128), jnp.float32),
    )(wet)


def _lane_sum(x):
    while x.shape[-1] > 1:
        h = x.shape[-1] // 2
        x = x[..., :h] + x[..., h:]
    return x


def _ln_body(we_ref, tt_ref, pe_ref, tte_ref, g_ref, b_ref, o_ref):
    bb, l, d = o_ref.shape
    we = we_ref[...].reshape(bb, l, we_ref.shape[-1])[:, :, :d]
    ttf = tt_ref[...].astype(jnp.float32)
    pe = pe_ref[...]
    tte = tte_ref[...]
    te = tte[0][None, None, :] + ttf[..., None] * (tte[1] - tte[0])[None, None, :]
    emb = we + pe[None] + te
    mean = jnp.mean(emb, axis=-1, keepdims=True)
    c = emb - mean
    var = jnp.mean(c * c, axis=-1, keepdims=True)
    inv = lax.rsqrt(var + _EPS)
    o_ref[...] = c * inv * g_ref[...][None, None, :] + b_ref[...][None, None, :]


def _tc_layernorm(we_pad, tt, pe, tte, gamma, beta):
    b, l = tt.shape
    d = pe.shape[-1]
    bb = 32
    grid = (b // bb,)
    return pl.pallas_call(
        _ln_body,
        grid=grid,
        in_specs=[
            pl.BlockSpec((bb * l, 128), lambda i: (i, 0)),
            pl.BlockSpec((bb, l), lambda i: (i, 0)),
            pl.BlockSpec((l, d), lambda i: (0, 0)),
            pl.BlockSpec((2, d), lambda i: (0, 0)),
            pl.BlockSpec((d,), lambda i: (0,)),
            pl.BlockSpec((d,), lambda i: (0,)),
        ],
        out_specs=pl.BlockSpec((bb, l, d), lambda i: (i, 0, 0)),
        out_shape=jax.ShapeDtypeStruct((b, l, d), jnp.float32),
    )(we_pad, tt, pe, tte, gamma, beta)


def kernel(input_ids, token_type_ids, word_emb, pos_emb, tok_type_emb, gamma, beta):
    b, l = input_ids.shape
    v, d = word_emb.shape
    n = b * l
    per_w = n // _NW
    ngroups = per_w // _GROUP
    ids = input_ids.astype(jnp.int32)
    table_pad = _tc_format(word_emb.T)
    ids3 = ids.reshape(_NW, ngroups, _GROUP)
    we_pad = _sc_gather(ids3, table_pad)
    return _tc_layernorm(we_pad, token_type_ids.astype(jnp.int32),
                         pos_emb[:l], tok_type_emb, gamma, beta)


# speedup vs baseline: 1.0731x; 1.0013x over previous
import functools

import jax
import jax.numpy as jnp
from jax import lax
from jax.experimental import pallas as pl
from jax.experimental.pallas import tpu as pltpu
from jax.experimental.pallas import tpu_sc as plsc

_EPS = 1e-12
_NC = 2
_NS = 16
_NW = _NC * _NS
_GROUP = 128


def _sc_gather(ids3, table):
    nw, ngroups, group = ids3.shape
    v, dp = table.shape
    n = nw * ngroups * group
    per_w = ngroups * group
    mesh = plsc.VectorSubcoreMesh(core_axis_name="c", subcore_axis_name="s")

    nbuf = 6
    nmain = ngroups // nbuf
    ntail = ngroups % nbuf

    @functools.partial(
        pl.kernel,
        mesh=mesh,
        out_type=jax.ShapeDtypeStruct((n, dp), jnp.float32),
        scratch_types=[
            pltpu.VMEM((ngroups, group), jnp.int32),
            pltpu.VMEM((nbuf, group, dp), jnp.float32),
            [pltpu.SemaphoreType.DMA] * nbuf,
        ],
        compiler_params=pltpu.CompilerParams(use_tc_tiling_on_sc=False),
    )
    def k(idx_hbm, table_hbm, out_hbm, idx_v, rows_v, sems):
        wid = lax.axis_index("s") * _NC + lax.axis_index("c")
        base = wid * per_w
        pltpu.sync_copy(idx_hbm.at[wid], idx_v)

        for s in range(nbuf):
            pltpu.async_copy(table_hbm.at[idx_v.at[s]], rows_v.at[s], sems[s])

        def body(jj, carry):
            j0 = jj * nbuf
            for s in range(nbuf):
                g = j0 + s
                pltpu.make_async_copy(table_hbm.at[idx_v.at[g]], rows_v.at[s],
                                      sems[s]).wait()
                off = pl.multiple_of(base + g * group, group)
                pltpu.sync_copy(rows_v.at[s], out_hbm.at[pl.ds(off, group)])

                @pl.when(g + nbuf < ngroups)
                def _():
                    pltpu.async_copy(table_hbm.at[idx_v.at[g + nbuf]],
                                     rows_v.at[s], sems[s])

            return carry

        lax.fori_loop(0, nmain, body, 0)

        for t in range(ntail):
            g = nmain * nbuf + t
            pltpu.make_async_copy(table_hbm.at[idx_v.at[g]], rows_v.at[t],
                                  sems[t]).wait()
            off = pl.multiple_of(base + g * group, group)
            pltpu.sync_copy(rows_v.at[t], out_hbm.at[pl.ds(off, group)])

    return k(ids3, table)


def _fmt_body(wet_ref, o_ref):
    x = wet_ref[...]
    o_ref[:, : x.shape[0]] = x.T


def _tc_format(wet, bi=20480):
    d, v = wet.shape
    nb = (v + bi - 1) // bi
    vp = nb * bi
    return pl.pallas_call(
        _fmt_body,
        grid=(nb,),
        in_specs=[pl.BlockSpec((d, bi), lambda k: (0, k))],
        out_specs=pl.BlockSpec((bi, 128), lambda k: (k, 0)),
        out_shape=jax.ShapeDtypeStruct((vp, 128), jnp.float32),
    )(wet)


def _ln_body(we_ref, tt_ref, pe_ref, tte_ref, g_ref, b_ref, o_ref):
    bb, l, d = o_ref.shape
    we = we_ref[...].reshape(bb, l, we_ref.shape[-1])[:, :, :d]
    ttf = tt_ref[...].astype(jnp.float32)
    pe = pe_ref[...]
    tte = tte_ref[...]
    te = tte[0][None, None, :] + ttf[..., None] * (tte[1] - tte[0])[None, None, :]
    emb = we + pe[None] + te
    mean = jnp.mean(emb, axis=-1, keepdims=True)
    c = emb - mean
    var = jnp.mean(c * c, axis=-1, keepdims=True)
    inv = lax.rsqrt(var + _EPS)
    o_ref[...] = c * inv * g_ref[...][None, None, :] + b_ref[...][None, None, :]


def _tc_layernorm(we_pad, tt, pe, tte, gamma, beta):
    b, l = tt.shape
    d = pe.shape[-1]
    bb = 32
    grid = (b // bb,)
    return pl.pallas_call(
        _ln_body,
        grid=grid,
        in_specs=[
            pl.BlockSpec((bb * l, 128), lambda i: (i, 0)),
            pl.BlockSpec((bb, l), lambda i: (i, 0)),
            pl.BlockSpec((l, d), lambda i: (0, 0)),
            pl.BlockSpec((2, d), lambda i: (0, 0)),
            pl.BlockSpec((d,), lambda i: (0,)),
            pl.BlockSpec((d,), lambda i: (0,)),
        ],
        out_specs=pl.BlockSpec((bb, l, d), lambda i: (i, 0, 0)),
        out_shape=jax.ShapeDtypeStruct((b, l, d), jnp.float32),
    )(we_pad, tt, pe, tte, gamma, beta)


def kernel(input_ids, token_type_ids, word_emb, pos_emb, tok_type_emb, gamma, beta):
    b, l = input_ids.shape
    v, d = word_emb.shape
    n = b * l
    per_w = n // _NW
    ngroups = per_w // _GROUP
    ids = input_ids.astype(jnp.int32)
    table_pad = _tc_format(word_emb.T)
    ids3 = ids.reshape(_NW, ngroups, _GROUP)
    we_pad = _sc_gather(ids3, table_pad)
    return _tc_layernorm(we_pad, token_type_ids.astype(jnp.int32),
                         pos_emb[:l], tok_type_emb, gamma, beta)


# speedup vs baseline: 1.0833x; 1.0095x over previous
import functools

import jax
import jax.numpy as jnp
from jax import lax
from jax.experimental import pallas as pl
from jax.experimental.pallas import tpu as pltpu
from jax.experimental.pallas import tpu_sc as plsc

_EPS = 1e-12
_NC = 2
_NS = 16
_NW = _NC * _NS
_GROUP = 128


def _sc_gather(ids3, table):
    nw, ngroups, group = ids3.shape
    v, dp = table.shape
    n = nw * ngroups * group
    per_w = ngroups * group
    mesh = plsc.VectorSubcoreMesh(core_axis_name="c", subcore_axis_name="s")

    nbuf = 6
    nmain = ngroups // nbuf
    ntail = ngroups % nbuf

    @functools.partial(
        pl.kernel,
        mesh=mesh,
        out_type=jax.ShapeDtypeStruct((n, dp), jnp.float32),
        scratch_types=[
            pltpu.VMEM((ngroups, group), jnp.int32),
            pltpu.VMEM((nbuf, group, dp), jnp.float32),
            [pltpu.SemaphoreType.DMA] * nbuf,
        ],
        compiler_params=pltpu.CompilerParams(use_tc_tiling_on_sc=False),
    )
    def k(idx_hbm, table_hbm, out_hbm, idx_v, rows_v, sems):
        wid = lax.axis_index("s") * _NC + lax.axis_index("c")
        base = wid * per_w
        pltpu.sync_copy(idx_hbm.at[wid], idx_v)

        for s in range(nbuf):
            pltpu.async_copy(table_hbm.at[idx_v.at[s]], rows_v.at[s], sems[s])

        def body(jj, carry):
            j0 = jj * nbuf
            for s in range(nbuf):
                g = j0 + s
                pltpu.make_async_copy(table_hbm.at[idx_v.at[g]], rows_v.at[s],
                                      sems[s]).wait()
                off = pl.multiple_of(base + g * group, group)
                pltpu.sync_copy(rows_v.at[s], out_hbm.at[pl.ds(off, group)])

                @pl.when(g + nbuf < ngroups)
                def _():
                    pltpu.async_copy(table_hbm.at[idx_v.at[g + nbuf]],
                                     rows_v.at[s], sems[s])

            return carry

        lax.fori_loop(0, nmain, body, 0)

        for t in range(ntail):
            g = nmain * nbuf + t
            pltpu.make_async_copy(table_hbm.at[idx_v.at[g]], rows_v.at[t],
                                  sems[t]).wait()
            off = pl.multiple_of(base + g * group, group)
            pltpu.sync_copy(rows_v.at[t], out_hbm.at[pl.ds(off, group)])

    return k(ids3, table)


def _fmt_body(wet_ref, o_ref):
    x = wet_ref[...]
    o_ref[:, : x.shape[0]] = x.T


def _tc_format(wet, bi=20480):
    d, v = wet.shape
    nb = (v + bi - 1) // bi
    vp = nb * bi
    return pl.pallas_call(
        _fmt_body,
        grid=(nb,),
        in_specs=[pl.BlockSpec((d, bi), lambda k: (0, k))],
        out_specs=pl.BlockSpec((bi, 128), lambda k: (k, 0)),
        out_shape=jax.ShapeDtypeStruct((vp, 128), jnp.float32),
    )(wet)


def _ln_body(we_ref, tt_ref, pe_ref, tte_ref, g_ref, b_ref, o_ref):
    bb, l, d = o_ref.shape
    we = we_ref[...].reshape(bb, l, we_ref.shape[-1])[:, :, :d]
    ttf = tt_ref[...].astype(jnp.float32)
    pe = pe_ref[...]
    tte = tte_ref[...]
    te = tte[0][None, None, :] + ttf[..., None] * (tte[1] - tte[0])[None, None, :]
    emb = we + pe[None] + te
    mean = jnp.mean(emb, axis=-1, keepdims=True)
    c = emb - mean
    var = jnp.mean(c * c, axis=-1, keepdims=True)
    inv = lax.rsqrt(var + _EPS)
    o_ref[...] = c * inv * g_ref[...][None, None, :] + b_ref[...][None, None, :]


def _tc_layernorm(we_pad, tt, pe, tte, gamma, beta):
    b, l = tt.shape
    d = pe.shape[-1]
    bb = 64
    grid = (b // bb,)
    return pl.pallas_call(
        _ln_body,
        grid=grid,
        in_specs=[
            pl.BlockSpec((bb * l, 128), lambda i: (i, 0)),
            pl.BlockSpec((bb, l), lambda i: (i, 0)),
            pl.BlockSpec((l, d), lambda i: (0, 0)),
            pl.BlockSpec((2, d), lambda i: (0, 0)),
            pl.BlockSpec((d,), lambda i: (0,)),
            pl.BlockSpec((d,), lambda i: (0,)),
        ],
        out_specs=pl.BlockSpec((bb, l, d), lambda i: (i, 0, 0)),
        out_shape=jax.ShapeDtypeStruct((b, l, d), jnp.float32),
    )(we_pad, tt, pe, tte, gamma, beta)


def kernel(input_ids, token_type_ids, word_emb, pos_emb, tok_type_emb, gamma, beta):
    b, l = input_ids.shape
    v, d = word_emb.shape
    n = b * l
    per_w = n // _NW
    ngroups = per_w // _GROUP
    ids = input_ids.astype(jnp.int32)
    table_pad = _tc_format(word_emb.T)
    ids3 = ids.reshape(_NW, ngroups, _GROUP)
    we_pad = _sc_gather(ids3, table_pad)
    return _tc_layernorm(we_pad, token_type_ids.astype(jnp.int32),
                         pos_emb[:l], tok_type_emb, gamma, beta)
